# initial kernel scaffold (unmeasured)
import jax
import jax.numpy as jnp
from jax import lax
from jax.experimental import pallas as pl
from jax.experimental.pallas import tpu as pltpu

N_DEV = 4
SQ_BLK = 256
D_MODEL = 1024
HQ_LOC = 8
DH = 128
KW = 512
WINDOW = 128
SCALE = 0.08838834764831843
K_STARTS = (0, 128, 384, 640)


def kernel(x, Wq, K_ext, V_ext, Wo):
    def body(x_ref, wq_ref, k_hbm, v_hbm, wo_ref, out_ref,
             xg, kbuf, vbuf, partials, rs_recv, rs_send,
             ag_send_sems, ag_recv_sems, rs_send_sems, rs_recv_sems,
             k_sems, v_sems):
        my = lax.axis_index("i")
        right = lax.rem(my + 1, N_DEV)
        left = lax.rem(my + N_DEV - 1, N_DEV)
        h0 = my * HQ_LOC

        kv_copies = []
        for s in range(N_DEV):
            ck = pltpu.make_async_copy(
                k_hbm.at[0, pl.ds(K_STARTS[s], KW), pl.ds(h0, HQ_LOC), :],
                kbuf.at[s], k_sems.at[s])
            cv = pltpu.make_async_copy(
                v_hbm.at[0, pl.ds(K_STARTS[s], KW), pl.ds(h0, HQ_LOC), :],
                vbuf.at[s], v_sems.at[s])
            ck.start()
            cv.start()
            kv_copies.append((ck, cv))

        bsem = pltpu.get_barrier_semaphore()
        for nbr in (left, right):
            pl.semaphore_signal(bsem, inc=1, device_id=(nbr,),
                                device_id_type=pl.DeviceIdType.MESH)
        pl.semaphore_wait(bsem, 2)

        pl.store(xg, (pl.ds(my, 1), slice(None), slice(None)),
                 x_ref[0].astype(jnp.bfloat16)[None])

        for h in range(N_DEV - 1):
            o = lax.rem(my - h + N_DEV, N_DEV)
            rdma = pltpu.make_async_remote_copy(
                src_ref=xg.at[o], dst_ref=xg.at[o],
                send_sem=ag_send_sems.at[h], recv_sem=ag_recv_sems.at[h],
                device_id=(right,), device_id_type=pl.DeviceIdType.MESH)
            rdma.start()
            rdma.wait()

        wq = wq_ref[...].astype(jnp.bfloat16)
        wo = wo_ref[...].astype(jnp.bfloat16)

        for s in range(N_DEV):
            xs = xg[s]
            q = jnp.dot(xs, wq, preferred_element_type=jnp.float32) * SCALE
            kv_copies[s][0].wait()
            kv_copies[s][1].wait()
            qg = s * SQ_BLK + lax.broadcasted_iota(jnp.int32, (SQ_BLK, KW), 0)
            kg = K_STARTS[s] + lax.broadcasted_iota(jnp.int32, (SQ_BLK, KW), 1)
            bias = jnp.where(jnp.abs(qg - kg) <= WINDOW, 0.0, -1e9).astype(
                jnp.float32)
            acc = jnp.zeros((SQ_BLK, D_MODEL), jnp.float32)
            for hh in range(HQ_LOC):
                qh = q[:, hh * DH:(hh + 1) * DH].astype(jnp.bfloat16)
                kh = kbuf[s, :, hh, :].astype(jnp.bfloat16)
                vh = vbuf[s, :, hh, :].astype(jnp.bfloat16)
                sc = lax.dot_general(qh, kh, (((1,), (1,)), ((), ())),
                                     preferred_element_type=jnp.float32)
                sc = sc + bias
                m = jnp.max(sc, axis=1, keepdims=True)
                w = jnp.exp(sc - m)
                p = (w / jnp.sum(w, axis=1, keepdims=True)).astype(jnp.bfloat16)
                ctx = jnp.dot(p, vh, preferred_element_type=jnp.float32)
                acc = acc + jnp.dot(ctx.astype(jnp.bfloat16),
                                    wo[hh * DH:(hh + 1) * DH, :],
                                    preferred_element_type=jnp.float32)
            partials[s] = acc

        c0 = lax.rem(my + N_DEV - 1, N_DEV)
        rs_send[...] = pl.load(
            partials, (pl.ds(c0, 1), slice(None), slice(None)))[0]
        for t in range(N_DEV - 1):
            rdma = pltpu.make_async_remote_copy(
                src_ref=rs_send, dst_ref=rs_recv.at[t],
                send_sem=rs_send_sems.at[t], recv_sem=rs_recv_sems.at[t],
                device_id=(right,), device_id_type=pl.DeviceIdType.MESH)
            rdma.start()
            rdma.wait()
            c = lax.rem(my + 2 * N_DEV - 2 - t, N_DEV)
            mine = pl.load(
                partials, (pl.ds(c, 1), slice(None), slice(None)))[0]
            tot = rs_recv[t] + mine
            if t < N_DEV - 2:
                rs_send[...] = tot
            else:
                out_ref[0] = tot

    out_shape = jax.ShapeDtypeStruct((1, SQ_BLK, D_MODEL), jnp.float32)
    return pl.pallas_call(
        body,
        out_shape=out_shape,
        in_specs=[
            pl.BlockSpec(memory_space=pltpu.VMEM),
            pl.BlockSpec(memory_space=pltpu.VMEM),
            pl.BlockSpec(memory_space=pltpu.ANY),
            pl.BlockSpec(memory_space=pltpu.ANY),
            pl.BlockSpec(memory_space=pltpu.VMEM),
        ],
        out_specs=pl.BlockSpec(memory_space=pltpu.VMEM),
        scratch_shapes=[
            pltpu.VMEM((N_DEV, SQ_BLK, D_MODEL), jnp.bfloat16),
            pltpu.VMEM((N_DEV, KW, HQ_LOC, DH), jnp.float32),
            pltpu.VMEM((N_DEV, KW, HQ_LOC, DH), jnp.float32),
            pltpu.VMEM((N_DEV, SQ_BLK, D_MODEL), jnp.float32),
            pltpu.VMEM((N_DEV - 1, SQ_BLK, D_MODEL), jnp.float32),
            pltpu.VMEM((SQ_BLK, D_MODEL), jnp.float32),
            pltpu.SemaphoreType.DMA((N_DEV - 1,)),
            pltpu.SemaphoreType.DMA((N_DEV - 1,)),
            pltpu.SemaphoreType.DMA((N_DEV - 1,)),
            pltpu.SemaphoreType.DMA((N_DEV - 1,)),
            pltpu.SemaphoreType.DMA((N_DEV,)),
            pltpu.SemaphoreType.DMA((N_DEV,)),
        ],
        compiler_params=pltpu.CompilerParams(
            collective_id=0,
            vmem_limit_bytes=100 * 1024 * 1024,
        ),
    )(x, Wq, K_ext, V_ext, Wo)


# baseline (device time: 108769 ns/iter reference)
import jax
import jax.numpy as jnp
from jax import lax
from jax.experimental import pallas as pl
from jax.experimental.pallas import tpu as pltpu

N_DEV = 4
SQ_BLK = 256
D_MODEL = 1024
HQ_LOC = 8
DH = 128
KW = 512
WINDOW = 128
SCALE = 0.08838834764831843
K_STARTS = (0, 128, 384, 640)


def kernel(x, Wq, K_ext, V_ext, Wo):
    def body(x_ref, wq_ref, k_hbm, v_hbm, wo_ref, out_ref,
             xg, kbuf, vbuf, partials, rs_recv, rs_send,
             ag_send_sems, ag_recv_sems, rs_send_sems, rs_recv_sems,
             k_sems, v_sems):
        my = lax.axis_index("i")
        right = lax.rem(my + 1, N_DEV)
        left = lax.rem(my + N_DEV - 1, N_DEV)
        h0 = my * HQ_LOC

        kv_copies = []
        for s in range(N_DEV):
            ck = pltpu.make_async_copy(
                k_hbm.at[0, pl.ds(K_STARTS[s], KW), pl.ds(h0, HQ_LOC), :],
                kbuf.at[s], k_sems.at[s])
            cv = pltpu.make_async_copy(
                v_hbm.at[0, pl.ds(K_STARTS[s], KW), pl.ds(h0, HQ_LOC), :],
                vbuf.at[s], v_sems.at[s])
            ck.start()
            cv.start()
            kv_copies.append((ck, cv))

        bsem = pltpu.get_barrier_semaphore()
        for nbr in (left, right):
            pl.semaphore_signal(bsem, inc=1, device_id=(nbr,),
                                device_id_type=pl.DeviceIdType.MESH)
        pl.semaphore_wait(bsem, 2)

        xg[pl.ds(my, 1)] = x_ref[0].astype(jnp.bfloat16)[None]

        for h in range(N_DEV - 1):
            o = lax.rem(my - h + N_DEV, N_DEV)
            rdma = pltpu.make_async_remote_copy(
                src_ref=xg.at[o], dst_ref=xg.at[o],
                send_sem=ag_send_sems.at[h], recv_sem=ag_recv_sems.at[h],
                device_id=(right,), device_id_type=pl.DeviceIdType.MESH)
            rdma.start()
            rdma.wait()

        wq = wq_ref[...].astype(jnp.bfloat16)
        wo = wo_ref[...].astype(jnp.bfloat16)

        for s in range(N_DEV):
            xs = xg[s]
            q = jnp.dot(xs, wq, preferred_element_type=jnp.float32) * SCALE
            kv_copies[s][0].wait()
            kv_copies[s][1].wait()
            qg = s * SQ_BLK + lax.broadcasted_iota(jnp.int32, (SQ_BLK, KW), 0)
            kg = K_STARTS[s] + lax.broadcasted_iota(jnp.int32, (SQ_BLK, KW), 1)
            bias = jnp.where(jnp.abs(qg - kg) <= WINDOW, 0.0, -1e9).astype(
                jnp.float32)
            acc = jnp.zeros((SQ_BLK, D_MODEL), jnp.float32)
            for hh in range(HQ_LOC):
                qh = q[:, hh * DH:(hh + 1) * DH].astype(jnp.bfloat16)
                kh = kbuf[s, :, hh, :].astype(jnp.bfloat16)
                vh = vbuf[s, :, hh, :].astype(jnp.bfloat16)
                sc = lax.dot_general(qh, kh, (((1,), (1,)), ((), ())),
                                     preferred_element_type=jnp.float32)
                sc = sc + bias
                m = jnp.max(sc, axis=1, keepdims=True)
                w = jnp.exp(sc - m)
                p = (w / jnp.sum(w, axis=1, keepdims=True)).astype(jnp.bfloat16)
                ctx = jnp.dot(p, vh, preferred_element_type=jnp.float32)
                acc = acc + jnp.dot(ctx.astype(jnp.bfloat16),
                                    wo[hh * DH:(hh + 1) * DH, :],
                                    preferred_element_type=jnp.float32)
            partials[s] = acc

        c0 = lax.rem(my + N_DEV - 1, N_DEV)
        rs_send[...] = partials[pl.ds(c0, 1)][0]
        for t in range(N_DEV - 1):
            rdma = pltpu.make_async_remote_copy(
                src_ref=rs_send, dst_ref=rs_recv.at[t],
                send_sem=rs_send_sems.at[t], recv_sem=rs_recv_sems.at[t],
                device_id=(right,), device_id_type=pl.DeviceIdType.MESH)
            rdma.start()
            rdma.wait()
            c = lax.rem(my + 2 * N_DEV - 2 - t, N_DEV)
            mine = partials[pl.ds(c, 1)][0]
            tot = rs_recv[t] + mine
            if t < N_DEV - 2:
                rs_send[...] = tot
            else:
                out_ref[0] = tot

    out_shape = jax.ShapeDtypeStruct((1, SQ_BLK, D_MODEL), jnp.float32)
    return pl.pallas_call(
        body,
        out_shape=out_shape,
        in_specs=[
            pl.BlockSpec(memory_space=pltpu.VMEM),
            pl.BlockSpec(memory_space=pltpu.VMEM),
            pl.BlockSpec(memory_space=pl.ANY),
            pl.BlockSpec(memory_space=pl.ANY),
            pl.BlockSpec(memory_space=pltpu.VMEM),
        ],
        out_specs=pl.BlockSpec(memory_space=pltpu.VMEM),
        scratch_shapes=[
            pltpu.VMEM((N_DEV, SQ_BLK, D_MODEL), jnp.bfloat16),
            pltpu.VMEM((N_DEV, KW, HQ_LOC, DH), jnp.float32),
            pltpu.VMEM((N_DEV, KW, HQ_LOC, DH), jnp.float32),
            pltpu.VMEM((N_DEV, SQ_BLK, D_MODEL), jnp.float32),
            pltpu.VMEM((N_DEV - 1, SQ_BLK, D_MODEL), jnp.float32),
            pltpu.VMEM((SQ_BLK, D_MODEL), jnp.float32),
            pltpu.SemaphoreType.DMA((N_DEV - 1,)),
            pltpu.SemaphoreType.DMA((N_DEV - 1,)),
            pltpu.SemaphoreType.DMA((N_DEV - 1,)),
            pltpu.SemaphoreType.DMA((N_DEV - 1,)),
            pltpu.SemaphoreType.DMA((N_DEV,)),
            pltpu.SemaphoreType.DMA((N_DEV,)),
        ],
        compiler_params=pltpu.CompilerParams(
            collective_id=0,
            vmem_limit_bytes=100 * 1024 * 1024,
        ),
    )(x, Wq, K_ext, V_ext, Wo)


# device time: 56956 ns/iter; 1.9097x vs baseline; 1.9097x over previous
import jax
import jax.numpy as jnp
from jax import lax
from jax.experimental import pallas as pl
from jax.experimental.pallas import tpu as pltpu

N_DEV = 4
SQ_BLK = 256
D_MODEL = 1024
HQ_LOC = 8
DH = 128
KW = 512
WINDOW = 128
SCALE = 0.08838834764831843
K_STARTS = (0, 128, 384, 640)


def kernel(x, Wq, K_ext, V_ext, Wo):
    def body(x_ref, wq_ref, k_hbm, v_hbm, wo_ref, out_ref,
             xg, kbuf, vbuf, rs_recv, rs_send,
             ag_send_sems, ag_recv_sems, rs_send_sems, rs_recv_sems,
             k_sems, v_sems):
        my = lax.axis_index("i")
        right = lax.rem(my + 1, N_DEV)
        left = lax.rem(my + N_DEV - 1, N_DEV)
        h0 = my * HQ_LOC

        for s in range(N_DEV):
            pltpu.make_async_copy(
                k_hbm.at[0, pl.ds(K_STARTS[s], KW), pl.ds(h0, HQ_LOC), :],
                kbuf.at[s], k_sems.at[s]).start()
            pltpu.make_async_copy(
                v_hbm.at[0, pl.ds(K_STARTS[s], KW), pl.ds(h0, HQ_LOC), :],
                vbuf.at[s], v_sems.at[s]).start()

        bsem = pltpu.get_barrier_semaphore()
        for nbr in (left, right):
            pl.semaphore_signal(bsem, inc=1, device_id=(nbr,),
                                device_id_type=pl.DeviceIdType.MESH)
        pl.semaphore_wait(bsem, 2)

        xg[pl.ds(my, 1)] = x_ref[0].astype(jnp.bfloat16)[None]

        wq = wq_ref[...].astype(jnp.bfloat16)
        wo = wo_ref[...].astype(jnp.bfloat16)

        def compute_block(o):
            kstart = jnp.maximum(o * SQ_BLK - WINDOW, 0)
            pltpu.make_async_copy(
                k_hbm.at[0, pl.ds(kstart, KW), pl.ds(h0, HQ_LOC), :],
                kbuf.at[o], k_sems.at[o]).wait()
            pltpu.make_async_copy(
                v_hbm.at[0, pl.ds(kstart, KW), pl.ds(h0, HQ_LOC), :],
                vbuf.at[o], v_sems.at[o]).wait()
            xs = xg[pl.ds(o, 1)][0]
            kb = kbuf[pl.ds(o, 1)][0]
            vb = vbuf[pl.ds(o, 1)][0]
            q = jnp.dot(xs, wq, preferred_element_type=jnp.float32) * SCALE
            qg = o * SQ_BLK + lax.broadcasted_iota(jnp.int32, (SQ_BLK, KW), 0)
            kg = kstart + lax.broadcasted_iota(jnp.int32, (SQ_BLK, KW), 1)
            bias = jnp.where(jnp.abs(qg - kg) <= WINDOW, 0.0, -1e9).astype(
                jnp.float32)
            acc = jnp.zeros((SQ_BLK, D_MODEL), jnp.float32)
            for hh in range(HQ_LOC):
                qh = q[:, hh * DH:(hh + 1) * DH].astype(jnp.bfloat16)
                kh = kb[:, hh, :].astype(jnp.bfloat16)
                vh = vb[:, hh, :].astype(jnp.bfloat16)
                sc = lax.dot_general(qh, kh, (((1,), (1,)), ((), ())),
                                     preferred_element_type=jnp.float32)
                sc = sc + bias
                m = jnp.max(sc, axis=1, keepdims=True)
                w = jnp.exp(sc - m)
                p = (w / jnp.sum(w, axis=1, keepdims=True)).astype(
                    jnp.bfloat16)
                ctx = jnp.dot(p, vh, preferred_element_type=jnp.float32)
                acc = acc + jnp.dot(ctx.astype(jnp.bfloat16),
                                    wo[hh * DH:(hh + 1) * DH, :],
                                    preferred_element_type=jnp.float32)
            return acc

        def ag_hop(h):
            o = lax.rem(my - h + N_DEV, N_DEV)
            d = pltpu.make_async_remote_copy(
                src_ref=xg.at[o], dst_ref=xg.at[o],
                send_sem=ag_send_sems.at[h], recv_sem=ag_recv_sems.at[h],
                device_id=(right,), device_id_type=pl.DeviceIdType.MESH)
            d.start()
            return d

        def rs_hop(t):
            d = pltpu.make_async_remote_copy(
                src_ref=rs_send.at[t], dst_ref=rs_recv.at[t],
                send_sem=rs_send_sems.at[t], recv_sem=rs_recv_sems.at[t],
                device_id=(right,), device_id_type=pl.DeviceIdType.MESH)
            d.start()
            return d

        d0 = ag_hop(0)
        p_own = compute_block(my)
        d0.wait_recv()

        d1 = ag_hop(1)
        o1 = lax.rem(my + 3, N_DEV)
        p1 = compute_block(o1)
        rs_send[0] = p1.astype(jnp.bfloat16)
        r0 = rs_hop(0)
        d1.wait_recv()

        d2 = ag_hop(2)
        o2 = lax.rem(my + 2, N_DEV)
        p2 = compute_block(o2)
        r0.wait_recv()
        rs_send[1] = (rs_recv[0].astype(jnp.float32) + p2).astype(jnp.bfloat16)
        r1 = rs_hop(1)
        d2.wait_recv()

        o3 = lax.rem(my + 1, N_DEV)
        p3 = compute_block(o3)
        r1.wait_recv()
        rs_send[2] = (rs_recv[1].astype(jnp.float32) + p3).astype(jnp.bfloat16)
        r2 = rs_hop(2)
        r2.wait_recv()
        out_ref[0] = rs_recv[2].astype(jnp.float32) + p_own

        for d in (d0, d1, d2, r0, r1, r2):
            d.wait_send()

    out_shape = jax.ShapeDtypeStruct((1, SQ_BLK, D_MODEL), jnp.float32)
    return pl.pallas_call(
        body,
        out_shape=out_shape,
        in_specs=[
            pl.BlockSpec(memory_space=pltpu.VMEM),
            pl.BlockSpec(memory_space=pltpu.VMEM),
            pl.BlockSpec(memory_space=pl.ANY),
            pl.BlockSpec(memory_space=pl.ANY),
            pl.BlockSpec(memory_space=pltpu.VMEM),
        ],
        out_specs=pl.BlockSpec(memory_space=pltpu.VMEM),
        scratch_shapes=[
            pltpu.VMEM((N_DEV, SQ_BLK, D_MODEL), jnp.bfloat16),
            pltpu.VMEM((N_DEV, KW, HQ_LOC, DH), jnp.float32),
            pltpu.VMEM((N_DEV, KW, HQ_LOC, DH), jnp.float32),
            pltpu.VMEM((N_DEV - 1, SQ_BLK, D_MODEL), jnp.bfloat16),
            pltpu.VMEM((N_DEV - 1, SQ_BLK, D_MODEL), jnp.bfloat16),
            pltpu.SemaphoreType.DMA((N_DEV - 1,)),
            pltpu.SemaphoreType.DMA((N_DEV - 1,)),
            pltpu.SemaphoreType.DMA((N_DEV - 1,)),
            pltpu.SemaphoreType.DMA((N_DEV - 1,)),
            pltpu.SemaphoreType.DMA((N_DEV,)),
            pltpu.SemaphoreType.DMA((N_DEV,)),
        ],
        compiler_params=pltpu.CompilerParams(
            collective_id=0,
            vmem_limit_bytes=100 * 1024 * 1024,
        ),
    )(x, Wq, K_ext, V_ext, Wo)


# device time: 55287 ns/iter; 1.9674x vs baseline; 1.0302x over previous
import jax
import jax.numpy as jnp
from jax import lax
from jax.experimental import pallas as pl
from jax.experimental.pallas import tpu as pltpu

N_DEV = 4
SQ_BLK = 256
D_MODEL = 1024
HQ_LOC = 8
DH = 128
KW = 512
WINDOW = 128
SCALE = 0.08838834764831843
K_STARTS = (0, 128, 384, 640)


def kernel(x, Wq, K_ext, V_ext, Wo):
    def body(x_ref, wq_ref, k_hbm, v_hbm, wo_ref, out_ref,
             xg, kbuf, vbuf, rs_recv, rs_send,
             ag_send_sems, ag_recv_sems, rs_send_sems, rs_recv_sems,
             k_sems, v_sems):
        my = lax.axis_index("i")
        right = lax.rem(my + 1, N_DEV)
        left = lax.rem(my + N_DEV - 1, N_DEV)
        h0 = my * HQ_LOC

        for s in range(N_DEV):
            pltpu.make_async_copy(
                k_hbm.at[0, pl.ds(K_STARTS[s], KW), pl.ds(h0, HQ_LOC), :],
                kbuf.at[s], k_sems.at[s]).start()
            pltpu.make_async_copy(
                v_hbm.at[0, pl.ds(K_STARTS[s], KW), pl.ds(h0, HQ_LOC), :],
                vbuf.at[s], v_sems.at[s]).start()

        bsem = pltpu.get_barrier_semaphore()
        for nbr in (left, right):
            pl.semaphore_signal(bsem, inc=1, device_id=(nbr,),
                                device_id_type=pl.DeviceIdType.MESH)
        pl.semaphore_wait(bsem, 2)

        xg[pl.ds(my, 1)] = x_ref[0].astype(jnp.bfloat16)[None]

        wq = (wq_ref[...] * SCALE).astype(jnp.bfloat16)
        wo = wo_ref[...].astype(jnp.bfloat16)

        def compute_block(o):
            kstart = jnp.maximum(o * SQ_BLK - WINDOW, 0)
            pltpu.make_async_copy(
                k_hbm.at[0, pl.ds(kstart, KW), pl.ds(h0, HQ_LOC), :],
                kbuf.at[o], k_sems.at[o]).wait()
            pltpu.make_async_copy(
                v_hbm.at[0, pl.ds(kstart, KW), pl.ds(h0, HQ_LOC), :],
                vbuf.at[o], v_sems.at[o]).wait()
            xs = xg[pl.ds(o, 1)][0]
            kb = kbuf[pl.ds(o, 1)][0]
            vb = vbuf[pl.ds(o, 1)][0]
            q = jnp.dot(xs, wq, preferred_element_type=jnp.float32).astype(
                jnp.bfloat16)
            qg = o * SQ_BLK + lax.broadcasted_iota(jnp.int32, (SQ_BLK, KW), 0)
            kg = kstart + lax.broadcasted_iota(jnp.int32, (SQ_BLK, KW), 1)
            bias = jnp.where(jnp.abs(qg - kg) <= WINDOW, 0.0, -1e9).astype(
                jnp.float32)
            ctxs = []
            for hh in range(HQ_LOC):
                qh = q[:, hh * DH:(hh + 1) * DH]
                kh = kb[:, hh, :].astype(jnp.bfloat16)
                vh = vb[:, hh, :].astype(jnp.bfloat16)
                sc = lax.dot_general(qh, kh, (((1,), (1,)), ((), ())),
                                     preferred_element_type=jnp.float32)
                sc = sc + bias
                m = jnp.max(sc, axis=1, keepdims=True)
                w = jnp.exp(sc - m)
                p = (w * (1.0 / jnp.sum(w, axis=1, keepdims=True))).astype(
                    jnp.bfloat16)
                ctxs.append(jnp.dot(p, vh, preferred_element_type=jnp.float32)
                            .astype(jnp.bfloat16))
            ctx = jnp.concatenate(ctxs, axis=1)
            return jnp.dot(ctx, wo, preferred_element_type=jnp.float32)

        def ag_hop(h):
            o = lax.rem(my - h + N_DEV, N_DEV)
            d = pltpu.make_async_remote_copy(
                src_ref=xg.at[o], dst_ref=xg.at[o],
                send_sem=ag_send_sems.at[h], recv_sem=ag_recv_sems.at[h],
                device_id=(right,), device_id_type=pl.DeviceIdType.MESH)
            d.start()
            return d

        def rs_hop(t):
            d = pltpu.make_async_remote_copy(
                src_ref=rs_send.at[t], dst_ref=rs_recv.at[t],
                send_sem=rs_send_sems.at[t], recv_sem=rs_recv_sems.at[t],
                device_id=(right,), device_id_type=pl.DeviceIdType.MESH)
            d.start()
            return d

        d0 = ag_hop(0)
        p_own = compute_block(my)
        d0.wait_recv()

        d1 = ag_hop(1)
        o1 = lax.rem(my + 3, N_DEV)
        p1 = compute_block(o1)
        rs_send[0] = p1.astype(jnp.bfloat16)
        r0 = rs_hop(0)
        d1.wait_recv()

        d2 = ag_hop(2)
        o2 = lax.rem(my + 2, N_DEV)
        p2 = compute_block(o2)
        r0.wait_recv()
        rs_send[1] = (rs_recv[0].astype(jnp.float32) + p2).astype(jnp.bfloat16)
        r1 = rs_hop(1)
        d2.wait_recv()

        o3 = lax.rem(my + 1, N_DEV)
        p3 = compute_block(o3)
        r1.wait_recv()
        rs_send[2] = (rs_recv[1].astype(jnp.float32) + p3).astype(jnp.bfloat16)
        r2 = rs_hop(2)
        r2.wait_recv()
        out_ref[0] = rs_recv[2].astype(jnp.float32) + p_own

        for d in (d0, d1, d2, r0, r1, r2):
            d.wait_send()

    out_shape = jax.ShapeDtypeStruct((1, SQ_BLK, D_MODEL), jnp.float32)
    return pl.pallas_call(
        body,
        out_shape=out_shape,
        in_specs=[
            pl.BlockSpec(memory_space=pltpu.VMEM),
            pl.BlockSpec(memory_space=pltpu.VMEM),
            pl.BlockSpec(memory_space=pl.ANY),
            pl.BlockSpec(memory_space=pl.ANY),
            pl.BlockSpec(memory_space=pltpu.VMEM),
        ],
        out_specs=pl.BlockSpec(memory_space=pltpu.VMEM),
        scratch_shapes=[
            pltpu.VMEM((N_DEV, SQ_BLK, D_MODEL), jnp.bfloat16),
            pltpu.VMEM((N_DEV, KW, HQ_LOC, DH), jnp.float32),
            pltpu.VMEM((N_DEV, KW, HQ_LOC, DH), jnp.float32),
            pltpu.VMEM((N_DEV - 1, SQ_BLK, D_MODEL), jnp.bfloat16),
            pltpu.VMEM((N_DEV - 1, SQ_BLK, D_MODEL), jnp.bfloat16),
            pltpu.SemaphoreType.DMA((N_DEV - 1,)),
            pltpu.SemaphoreType.DMA((N_DEV - 1,)),
            pltpu.SemaphoreType.DMA((N_DEV - 1,)),
            pltpu.SemaphoreType.DMA((N_DEV - 1,)),
            pltpu.SemaphoreType.DMA((N_DEV,)),
            pltpu.SemaphoreType.DMA((N_DEV,)),
        ],
        compiler_params=pltpu.CompilerParams(
            collective_id=0,
            vmem_limit_bytes=100 * 1024 * 1024,
        ),
    )(x, Wq, K_ext, V_ext, Wo)


# device time: 38746 ns/iter; 2.8072x vs baseline; 1.4269x over previous
import functools

import jax
import jax.numpy as jnp
from jax import lax
from jax.experimental import pallas as pl
from jax.experimental.pallas import tpu as pltpu

N_DEV = 4
SQ_BLK = 256
D_MODEL = 1024
HQ_LOC = 8
DH = 128
KW = 512
KU = 1152
WINDOW = 128
SCALE = 0.08838834764831843


def kernel(x, Wq, K_ext, V_ext, Wo):
    def body(x_ref, wq_ref, k_hbm, v_hbm, wo_ref, out_ref,
             xg, kall, vall, kallb, vallb, pout, pin,
             ag_send_sems, ag_recv_sems, p_send_sems, p_recv_sems,
             k_sem, v_sem):
        my = lax.axis_index("i")
        right = lax.rem(my + 1, N_DEV)
        left = lax.rem(my + N_DEV - 1, N_DEV)
        opposite = lax.rem(my + 2, N_DEV)
        h0 = my * HQ_LOC

        for hh in range(HQ_LOC):
            pltpu.make_async_copy(
                k_hbm.at[0, pl.ds(0, KU), h0 + hh, :], kall.at[hh],
                k_sem).start()
            pltpu.make_async_copy(
                v_hbm.at[0, pl.ds(0, KU), h0 + hh, :], vall.at[hh],
                v_sem).start()

        bsem = pltpu.get_barrier_semaphore()
        for nbr in (left, right):
            pl.semaphore_signal(bsem, inc=1, device_id=(nbr,),
                                device_id_type=pl.DeviceIdType.MESH)
        pl.semaphore_wait(bsem, 2)

        xg[pl.ds(my, 1)] = x_ref[0].astype(jnp.bfloat16)[None]

        def ag(i, o, dev):
            d = pltpu.make_async_remote_copy(
                src_ref=xg.at[o], dst_ref=xg.at[o],
                send_sem=ag_send_sems.at[i], recv_sem=ag_recv_sems.at[i],
                device_id=(dev,), device_id_type=pl.DeviceIdType.MESH)
            d.start()
            return d

        def p_send(k, dev):
            d = pltpu.make_async_remote_copy(
                src_ref=pout.at[k], dst_ref=pin.at[k],
                send_sem=p_send_sems.at[k], recv_sem=p_recv_sems.at[k],
                device_id=(dev,), device_id_type=pl.DeviceIdType.MESH)
            d.start()
            return d

        a0r = ag(0, my, right)
        a0l = ag(1, my, left)

        wq = (wq_ref[...] * SCALE).astype(jnp.bfloat16)
        wo = wo_ref[...].astype(jnp.bfloat16)

        for hh in range(HQ_LOC):
            pltpu.make_async_copy(
                k_hbm.at[0, pl.ds(0, KU), h0 + hh, :], kall.at[hh],
                k_sem).wait()
            pltpu.make_async_copy(
                v_hbm.at[0, pl.ds(0, KU), h0 + hh, :], vall.at[hh],
                v_sem).wait()
        kallb[...] = kall[...].astype(jnp.bfloat16)
        vallb[...] = vall[...].astype(jnp.bfloat16)

        ii = lax.broadcasted_iota(jnp.int32, (SQ_BLK, KW), 0)
        jj = lax.broadcasted_iota(jnp.int32, (SQ_BLK, KW), 1)
        bias0 = jnp.where(jnp.abs(ii - jj) <= WINDOW, 0.0, -1e9).astype(
            jnp.float32)
        biasm = jnp.where(jnp.abs(ii - jj + WINDOW) <= WINDOW, 0.0,
                          -1e9).astype(jnp.float32)

        def compute_block(o):
            kstart = pl.multiple_of(jnp.maximum(o * SQ_BLK - WINDOW, 0), 128)
            xs = xg[pl.ds(o, 1)][0]
            q = jnp.dot(xs, wq, preferred_element_type=jnp.float32).astype(
                jnp.bfloat16)
            bias = jnp.where(o == 0, bias0, biasm)
            ctxs = []
            for hh in range(HQ_LOC):
                qh = q[:, hh * DH:(hh + 1) * DH]
                kh = kallb[hh, pl.ds(kstart, KW), :]
                vh = vallb[hh, pl.ds(kstart, KW), :]
                sc = lax.dot_general(qh, kh, (((1,), (1,)), ((), ())),
                                     preferred_element_type=jnp.float32)
                p = jnp.exp(sc + bias).astype(jnp.bfloat16)
                s = jnp.sum(p, axis=1, keepdims=True,
                            dtype=jnp.float32)
                ctx = jnp.dot(p, vh, preferred_element_type=jnp.float32)
                ctxs.append((ctx * (1.0 / s)).astype(jnp.bfloat16))
            ctx = jnp.concatenate(ctxs, axis=1)
            return jnp.dot(ctx, wo, preferred_element_type=jnp.float32)

        p_own = compute_block(my)
        a0r.wait_recv()
        a1 = ag(2, lax.rem(my + 3, N_DEV), right)
        a0l.wait_recv()

        pout[0] = compute_block(lax.rem(my + 1, N_DEV)).astype(jnp.bfloat16)
        pr = p_send(0, right)
        pout[1] = compute_block(lax.rem(my + 3, N_DEV)).astype(jnp.bfloat16)
        plft = p_send(1, left)
        a1.wait_recv()
        pout[2] = compute_block(opposite).astype(jnp.bfloat16)
        p2 = p_send(2, opposite)

        pr.wait_recv()
        plft.wait_recv()
        p2.wait_recv()
        out_ref[0] = (p_own + pin[0].astype(jnp.float32)
                      + pin[1].astype(jnp.float32)
                      + pin[2].astype(jnp.float32))

        for d in (a0r, a0l, a1, pr, plft, p2):
            d.wait_send()

        @functools.partial(pl.run_scoped,
                           second_barrier=pltpu.SemaphoreType.REGULAR)
        def _(second_barrier):
            for nbr in (left, right):
                pl.semaphore_signal(second_barrier, inc=1, device_id=(nbr,),
                                    device_id_type=pl.DeviceIdType.MESH)
            pl.semaphore_wait(second_barrier, 2)

    out_shape = jax.ShapeDtypeStruct((1, SQ_BLK, D_MODEL), jnp.float32)
    return pl.pallas_call(
        body,
        out_shape=out_shape,
        in_specs=[
            pl.BlockSpec(memory_space=pltpu.VMEM),
            pl.BlockSpec(memory_space=pltpu.VMEM),
            pl.BlockSpec(memory_space=pl.ANY),
            pl.BlockSpec(memory_space=pl.ANY),
            pl.BlockSpec(memory_space=pltpu.VMEM),
        ],
        out_specs=pl.BlockSpec(memory_space=pltpu.VMEM),
        scratch_shapes=[
            pltpu.VMEM((N_DEV, SQ_BLK, D_MODEL), jnp.bfloat16),
            pltpu.VMEM((HQ_LOC, KU, DH), jnp.float32),
            pltpu.VMEM((HQ_LOC, KU, DH), jnp.float32),
            pltpu.VMEM((HQ_LOC, KU, DH), jnp.bfloat16),
            pltpu.VMEM((HQ_LOC, KU, DH), jnp.bfloat16),
            pltpu.VMEM((N_DEV - 1, SQ_BLK, D_MODEL), jnp.bfloat16),
            pltpu.VMEM((N_DEV - 1, SQ_BLK, D_MODEL), jnp.bfloat16),
            pltpu.SemaphoreType.DMA((N_DEV - 1,)),
            pltpu.SemaphoreType.DMA((N_DEV - 1,)),
            pltpu.SemaphoreType.DMA((N_DEV - 1,)),
            pltpu.SemaphoreType.DMA((N_DEV - 1,)),
            pltpu.SemaphoreType.DMA,
            pltpu.SemaphoreType.DMA,
        ],
        compiler_params=pltpu.CompilerParams(
            collective_id=0,
            vmem_limit_bytes=100 * 1024 * 1024,
        ),
    )(x, Wq, K_ext, V_ext, Wo)
